# baseline (speedup 1.0000x reference)
import functools

import jax
import jax.numpy as jnp
from jax import lax
from jax.experimental import pallas as pl
from jax.experimental.pallas import tpu as pltpu
from jax.experimental.pallas import tpu_sc as plsc

_B = 256
_L = 1000
_D = 128
_H = 256
_NW = 32
_LPW = 32
_NS = _D // 16

_BSPLIT = 128
_BSC = _B - _BSPLIT
_BCH = 32
_NCH = _BSC // _BCH
_NT = _NCH * (_LPW // 8)
_NB = 8
_TC_STEPS = _BSPLIT // _NB


def _lane_bcast(vec, j):
    idx = jnp.full((16,), j, jnp.int32)
    return lax.gather(
        vec, idx[:, None],
        dimension_numbers=lax.GatherDimensionNumbers(
            offset_dims=(), collapsed_slice_dims=(0,), start_index_map=(0,)),
        slice_sizes=(1,), mode=lax.GatherScatterMode.PROMISE_IN_BOUNDS)


def _sc_reduce_body(x_hbm, maskT_hbm, pos_hbm, tot_hbm,
                    xbuf0, xbuf1, mvmem, pstage, tstage, sem0, sem1):
    cid = lax.axis_index("c")
    sid = lax.axis_index("s")
    wid = sid * 2 + cid
    l0 = jnp.minimum(wid * _LPW, _L - _LPW)

    pltpu.sync_copy(maskT_hbm.at[pl.ds(l0, _LPW)], mvmem)

    zero = jnp.zeros((16,), jnp.float32)

    def zrow(r, _):
        for s in range(_NS):
            pstage[r, pl.ds(s * 16, 16)] = zero
            tstage[r, pl.ds(s * 16, 16)] = zero
        return _

    lax.fori_loop(0, _LPW, zrow, None)

    def src(t):
        g = lax.div(t, _NCH)
        c = lax.rem(t, _NCH)
        return x_hbm.at[pl.ds(_BSPLIT + c * _BCH, _BCH), pl.ds(l0 + g * 8, 8)]

    pltpu.async_copy(src(0), xbuf0, sem0)
    pltpu.async_copy(src(1), xbuf1, sem1)

    def compute(t, buf):
        g = lax.div(t, _NCH)
        c = lax.rem(t, _NCH)
        for l in range(8):
            row = g * 8 + l

            def bbody(bl, carry, _l=l):
                b = c * _BCH + bl
                j = lax.rem(b, 16)
                mrow = mvmem[row, pl.ds(b - j, 16)]
                m = _lane_bcast(mrow, j)
                out = list(carry)
                for s in range(_NS):
                    xs = buf[bl, _l, pl.ds(s * 16, 16)]
                    out[s] = out[s] + m * xs
                    out[_NS + s] = out[_NS + s] + xs
                return tuple(out)

            acc = lax.fori_loop(0, _BCH, bbody, (zero,) * (2 * _NS))
            for s in range(_NS):
                plsc.addupdate(pstage.at[row, pl.ds(s * 16, 16)], acc[s])
                plsc.addupdate(tstage.at[row, pl.ds(s * 16, 16)],
                               acc[_NS + s])

    def pair(k, _):
        t = 2 * k
        pltpu.make_async_copy(src(t), xbuf0, sem0).wait()
        compute(t, xbuf0)

        @pl.when(t + 2 < _NT)
        def _pf0():
            pltpu.async_copy(src(t + 2), xbuf0, sem0)

        pltpu.make_async_copy(src(t), xbuf1, sem1).wait()
        compute(t + 1, xbuf1)

        @pl.when(t + 3 < _NT)
        def _pf1():
            pltpu.async_copy(src(t + 3), xbuf1, sem1)

        return _

    lax.fori_loop(0, _NT // 2, pair, None)

    pltpu.sync_copy(pstage, pos_hbm.at[pl.ds(l0, _LPW)])
    pltpu.sync_copy(tstage, tot_hbm.at[pl.ds(l0, _LPW)])


_sc_reduce = functools.partial(
    pl.kernel,
    out_type=[
        jax.ShapeDtypeStruct((_L, _D), jnp.float32),
        jax.ShapeDtypeStruct((_L, _D), jnp.float32),
    ],
    mesh=plsc.VectorSubcoreMesh(core_axis_name="c", subcore_axis_name="s"),
    scratch_types=[
        pltpu.VMEM((_BCH, 8, _D), jnp.float32),
        pltpu.VMEM((_BCH, 8, _D), jnp.float32),
        pltpu.VMEM((_LPW, _BSC), jnp.float32),
        pltpu.VMEM((_LPW, _D), jnp.float32),
        pltpu.VMEM((_LPW, _D), jnp.float32),
        pltpu.SemaphoreType.DMA,
        pltpu.SemaphoreType.DMA,
    ],
)(_sc_reduce_body)


def _tc_reduce_body(x_ref, mask_ref, pos_ref, tot_ref, pos_acc, tot_acc):
    i = pl.program_id(0)

    @pl.when(i == 0)
    def _init():
        pos_acc[...] = jnp.zeros_like(pos_acc)
        tot_acc[...] = jnp.zeros_like(tot_acc)

    xb = x_ref[...]
    mb = mask_ref[...].astype(jnp.float32)
    pos_acc[...] += jnp.sum(mb[:, :, None] * xb, axis=0)
    tot_acc[...] += jnp.sum(xb, axis=0)

    @pl.when(i == _TC_STEPS - 1)
    def _finish():
        pos_ref[...] = pos_acc[...]
        tot_ref[...] = tot_acc[...]


def _tc_reduce(x, label_mask):
    return pl.pallas_call(
        _tc_reduce_body,
        grid=(_TC_STEPS,),
        in_specs=[
            pl.BlockSpec((_NB, _L, _D), lambda i: (i, 0, 0)),
            pl.BlockSpec((_NB, _L), lambda i: (i, 0)),
        ],
        out_specs=[
            pl.BlockSpec((_L, _D), lambda i: (0, 0)),
            pl.BlockSpec((_L, _D), lambda i: (0, 0)),
        ],
        out_shape=[
            jax.ShapeDtypeStruct((_L, _D), jnp.float32),
            jax.ShapeDtypeStruct((_L, _D), jnp.float32),
        ],
        scratch_shapes=[
            pltpu.VMEM((_L, _D), jnp.float32),
            pltpu.VMEM((_L, _D), jnp.float32),
        ],
        compiler_params=pltpu.CompilerParams(
            dimension_semantics=("arbitrary",),
        ),
    )(x, label_mask)


def _mlp(v, wh, bh, wp, bp):
    h = jnp.maximum(
        lax.dot_general(v, wh, (((1,), (1,)), ((), ())),
                        preferred_element_type=jnp.float32) + bh,
        0.0)
    return lax.dot_general(h, wp, (((1,), (1,)), ((), ())),
                           preferred_element_type=jnp.float32) + bp


def _tc_finish_body(pos1_ref, tot1_ref, pos2_ref, tot2_ref, maskT_ref,
                    gpt_ref, gpt_last_ref, wh_ref, bh_ref, wp_ref, bp_ref,
                    out_ref):
    cnt = jnp.sum(maskT_ref[...], axis=1, keepdims=True)
    pos = pos1_ref[...] + pos2_ref[...]
    tot = tot1_ref[...] + tot2_ref[...]
    neg_cnt = _B - cnt
    pos_mean = pos / jnp.maximum(cnt, 1.0)
    neg_mean = (tot - pos) / jnp.maximum(neg_cnt, 1.0)
    avg = 0.5 * pos_mean + 0.5 * gpt_ref[...]
    avg_anti = 0.5 * neg_mean + 0.5 * gpt_last_ref[...]
    wh = wh_ref[...]
    bh = bh_ref[...]
    wp = wp_ref[...]
    bp = bp_ref[...]
    proto = _mlp(avg, wh, bh, wp, bp)
    proto = jnp.where(cnt > 0.0, proto, 0.0)
    anti = _mlp(avg_anti, wh, bh, wp, bp)
    valid = (neg_cnt > 0.0).astype(jnp.float32)
    anti_sum = jnp.sum(anti * valid, axis=0, keepdims=True)
    anti_row = anti_sum / jnp.maximum(jnp.sum(valid), 1.0)
    out_ref[0:_L, :] = proto
    out_ref[_L:_L + 1, :] = anti_row


def _tc_finish(pos1, tot1, pos2, tot2, maskT, gpt_main, gpt_last,
               wh, bh, wp, bp):
    return pl.pallas_call(
        _tc_finish_body,
        out_shape=jax.ShapeDtypeStruct((_L + 1, _D), jnp.float32),
    )(pos1, tot1, pos2, tot2, maskT, gpt_main, gpt_last, wh, bh, wp, bp)


@jax.jit
def _run(x, label_mask, gpt_main, gpt_last, wh, bh, wp, bp):
    maskT = jnp.transpose(label_mask, (1, 0)).astype(jnp.float32)
    pos_sc, tot_sc = _sc_reduce(x, maskT[:, _BSPLIT:])
    pos_tc, tot_tc = _tc_reduce(x, label_mask)
    return _tc_finish(pos_tc, tot_tc, pos_sc, tot_sc, maskT,
                      gpt_main, gpt_last, wh, bh, wp, bp)


def kernel(x, label_mask, global_prototype_tensor, W_hidden, b_hidden,
           W_proto, b_proto):
    gpt_main = global_prototype_tensor[:_L]
    gpt_last = global_prototype_tensor[_L:]
    return _run(x, label_mask, gpt_main, gpt_last, W_hidden,
                b_hidden.reshape(1, _H), W_proto, b_proto.reshape(1, _D))


# speedup vs baseline: 1.0202x; 1.0202x over previous
import functools

import jax
import jax.numpy as jnp
from jax import lax
from jax.experimental import pallas as pl
from jax.experimental.pallas import tpu as pltpu
from jax.experimental.pallas import tpu_sc as plsc

_B = 256
_L = 1000
_D = 128
_H = 256
_NW = 32
_LPW = 32
_NS = _D // 16

_BSPLIT = 128
_BSC = _B - _BSPLIT
_NB = 8
_TC_STEPS = _BSPLIT // _NB


def _lane_bcast(vec, j):
    idx = jnp.full((16,), j, jnp.int32)
    return lax.gather(
        vec, idx[:, None],
        dimension_numbers=lax.GatherDimensionNumbers(
            offset_dims=(), collapsed_slice_dims=(0,), start_index_map=(0,)),
        slice_sizes=(1,), mode=lax.GatherScatterMode.PROMISE_IN_BOUNDS)


def _sc_reduce_body(x_hbm, maskT_hbm, pos_hbm, tot_hbm,
                    xbuf0, xbuf1, mvmem, pstage, tstage, sem0, sem1):
    cid = lax.axis_index("c")
    sid = lax.axis_index("s")
    wid = sid * 2 + cid
    l0 = jnp.minimum(wid * _LPW, _L - _LPW)

    pltpu.sync_copy(maskT_hbm.at[pl.ds(l0, _LPW)], mvmem)

    pltpu.async_copy(x_hbm.at[pl.ds(_BSPLIT, _BSC), l0], xbuf0, sem0)
    pltpu.async_copy(x_hbm.at[pl.ds(_BSPLIT, _BSC), l0 + 1], xbuf1, sem1)

    zero = jnp.zeros((16,), jnp.float32)

    def compute(l, buf):
        def bbody(b, carry):
            j = lax.rem(b, 16)
            mrow = mvmem[l, pl.ds(b - j, 16)]
            m = _lane_bcast(mrow, j)
            out = list(carry)
            for s in range(_NS):
                xs = buf[b, pl.ds(s * 16, 16)]
                out[s] = out[s] + m * xs
                out[_NS + s] = out[_NS + s] + xs
            return tuple(out)

        acc = lax.fori_loop(0, _BSC, bbody, (zero,) * (2 * _NS),
                            unroll=4)
        for s in range(_NS):
            pstage[l, pl.ds(s * 16, 16)] = acc[s]
            tstage[l, pl.ds(s * 16, 16)] = acc[_NS + s]

    def pair(k, _):
        l = 2 * k
        pltpu.make_async_copy(x_hbm.at[pl.ds(_BSPLIT, _BSC), l0],
                              xbuf0, sem0).wait()
        compute(l, xbuf0)

        @pl.when(l + 2 < _LPW)
        def _pf0():
            pltpu.async_copy(x_hbm.at[pl.ds(_BSPLIT, _BSC), l0 + l + 2],
                             xbuf0, sem0)

        pltpu.make_async_copy(x_hbm.at[pl.ds(_BSPLIT, _BSC), l0],
                              xbuf1, sem1).wait()
        compute(l + 1, xbuf1)

        @pl.when(l + 3 < _LPW)
        def _pf1():
            pltpu.async_copy(x_hbm.at[pl.ds(_BSPLIT, _BSC), l0 + l + 3],
                             xbuf1, sem1)

        return _

    lax.fori_loop(0, _LPW // 2, pair, None)

    pltpu.sync_copy(pstage, pos_hbm.at[pl.ds(l0, _LPW)])
    pltpu.sync_copy(tstage, tot_hbm.at[pl.ds(l0, _LPW)])


_sc_reduce = functools.partial(
    pl.kernel,
    out_type=[
        jax.ShapeDtypeStruct((_L, _D), jnp.float32),
        jax.ShapeDtypeStruct((_L, _D), jnp.float32),
    ],
    mesh=plsc.VectorSubcoreMesh(core_axis_name="c", subcore_axis_name="s"),
    scratch_types=[
        pltpu.VMEM((_BSC, _D), jnp.float32),
        pltpu.VMEM((_BSC, _D), jnp.float32),
        pltpu.VMEM((_LPW, _BSC), jnp.float32),
        pltpu.VMEM((_LPW, _D), jnp.float32),
        pltpu.VMEM((_LPW, _D), jnp.float32),
        pltpu.SemaphoreType.DMA,
        pltpu.SemaphoreType.DMA,
    ],
)(_sc_reduce_body)


def _tc_reduce_body(x_ref, mask_ref, pos_ref, tot_ref, pos_acc, tot_acc):
    i = pl.program_id(0)

    @pl.when(i == 0)
    def _init():
        pos_acc[...] = jnp.zeros_like(pos_acc)
        tot_acc[...] = jnp.zeros_like(tot_acc)

    xb = x_ref[...]
    mb = mask_ref[...].astype(jnp.float32)
    pos_acc[...] += jnp.sum(mb[:, :, None] * xb, axis=0)
    tot_acc[...] += jnp.sum(xb, axis=0)

    @pl.when(i == _TC_STEPS - 1)
    def _finish():
        pos_ref[...] = pos_acc[...]
        tot_ref[...] = tot_acc[...]


def _tc_reduce(x, label_mask):
    return pl.pallas_call(
        _tc_reduce_body,
        grid=(_TC_STEPS,),
        in_specs=[
            pl.BlockSpec((_NB, _L, _D), lambda i: (i, 0, 0)),
            pl.BlockSpec((_NB, _L), lambda i: (i, 0)),
        ],
        out_specs=[
            pl.BlockSpec((_L, _D), lambda i: (0, 0)),
            pl.BlockSpec((_L, _D), lambda i: (0, 0)),
        ],
        out_shape=[
            jax.ShapeDtypeStruct((_L, _D), jnp.float32),
            jax.ShapeDtypeStruct((_L, _D), jnp.float32),
        ],
        scratch_shapes=[
            pltpu.VMEM((_L, _D), jnp.float32),
            pltpu.VMEM((_L, _D), jnp.float32),
        ],
        compiler_params=pltpu.CompilerParams(
            dimension_semantics=("arbitrary",),
        ),
    )(x, label_mask)


def _mlp(v, wh, bh, wp, bp):
    h = jnp.maximum(
        lax.dot_general(v, wh, (((1,), (1,)), ((), ())),
                        preferred_element_type=jnp.float32) + bh,
        0.0)
    return lax.dot_general(h, wp, (((1,), (1,)), ((), ())),
                           preferred_element_type=jnp.float32) + bp


def _tc_finish_body(pos1_ref, tot1_ref, pos2_ref, tot2_ref, maskT_ref,
                    gpt_ref, gpt_last_ref, wh_ref, bh_ref, wp_ref, bp_ref,
                    out_ref):
    cnt = jnp.sum(maskT_ref[...], axis=1, keepdims=True)
    pos = pos1_ref[...] + pos2_ref[...]
    tot = tot1_ref[...] + tot2_ref[...]
    neg_cnt = _B - cnt
    pos_mean = pos / jnp.maximum(cnt, 1.0)
    neg_mean = (tot - pos) / jnp.maximum(neg_cnt, 1.0)
    avg = 0.5 * pos_mean + 0.5 * gpt_ref[...]
    avg_anti = 0.5 * neg_mean + 0.5 * gpt_last_ref[...]
    wh = wh_ref[...]
    bh = bh_ref[...]
    wp = wp_ref[...]
    bp = bp_ref[...]
    proto = _mlp(avg, wh, bh, wp, bp)
    proto = jnp.where(cnt > 0.0, proto, 0.0)
    anti = _mlp(avg_anti, wh, bh, wp, bp)
    valid = (neg_cnt > 0.0).astype(jnp.float32)
    anti_sum = jnp.sum(anti * valid, axis=0, keepdims=True)
    anti_row = anti_sum / jnp.maximum(jnp.sum(valid), 1.0)
    out_ref[0:_L, :] = proto
    out_ref[_L:_L + 1, :] = anti_row


def _tc_finish(pos1, tot1, pos2, tot2, maskT, gpt_main, gpt_last,
               wh, bh, wp, bp):
    return pl.pallas_call(
        _tc_finish_body,
        out_shape=jax.ShapeDtypeStruct((_L + 1, _D), jnp.float32),
    )(pos1, tot1, pos2, tot2, maskT, gpt_main, gpt_last, wh, bh, wp, bp)


@jax.jit
def _run(x, label_mask, gpt_main, gpt_last, wh, bh, wp, bp):
    maskT = jnp.transpose(label_mask, (1, 0)).astype(jnp.float32)
    pos_sc, tot_sc = _sc_reduce(x, maskT[:, _BSPLIT:])
    pos_tc, tot_tc = _tc_reduce(x, label_mask)
    return _tc_finish(pos_tc, tot_tc, pos_sc, tot_sc, maskT,
                      gpt_main, gpt_last, wh, bh, wp, bp)


def kernel(x, label_mask, global_prototype_tensor, W_hidden, b_hidden,
           W_proto, b_proto):
    gpt_main = global_prototype_tensor[:_L]
    gpt_last = global_prototype_tensor[_L:]
    return _run(x, label_mask, gpt_main, gpt_last, W_hidden,
                b_hidden.reshape(1, _H), W_proto, b_proto.reshape(1, _D))


# speedup vs baseline: 1.0310x; 1.0106x over previous
import functools

import jax
import jax.numpy as jnp
from jax import lax
from jax.experimental import pallas as pl
from jax.experimental.pallas import tpu as pltpu
from jax.experimental.pallas import tpu_sc as plsc

_B = 256
_L = 1000
_D = 128
_H = 256
_NW = 32
_LPW = 32
_NS = _D // 16

_BSPLIT = 128
_BSC = _B - _BSPLIT
_NB = 8
_TC_STEPS = _BSPLIT // _NB


def _lane_bcast(vec, j):
    idx = jnp.full((16,), j, jnp.int32)
    return lax.gather(
        vec, idx[:, None],
        dimension_numbers=lax.GatherDimensionNumbers(
            offset_dims=(), collapsed_slice_dims=(0,), start_index_map=(0,)),
        slice_sizes=(1,), mode=lax.GatherScatterMode.PROMISE_IN_BOUNDS)


def _sc_reduce_body(x_hbm, maskT_hbm, pos_hbm, tot_hbm,
                    xbuf0, xbuf1, mvmem, pstage, tstage, sem0, sem1):
    cid = lax.axis_index("c")
    sid = lax.axis_index("s")
    wid = sid * 2 + cid
    l0 = jnp.minimum(wid * _LPW, _L - _LPW)

    pltpu.async_copy(x_hbm.at[pl.ds(_BSPLIT, _BSC), l0], xbuf0, sem0)
    pltpu.async_copy(x_hbm.at[pl.ds(_BSPLIT, _BSC), l0 + 1], xbuf1, sem1)

    pltpu.sync_copy(maskT_hbm.at[pl.ds(l0, _LPW)], mvmem)

    zero = jnp.zeros((16,), jnp.float32)

    def compute(l, buf):
        def bbody(b, carry):
            j = lax.rem(b, 16)
            mrow = mvmem[l, pl.ds(b - j, 16)]
            m = _lane_bcast(mrow, j)
            out = list(carry)
            for s in range(_NS):
                xs = buf[b, pl.ds(s * 16, 16)]
                out[s] = out[s] + m * xs
                out[_NS + s] = out[_NS + s] + xs
            return tuple(out)

        acc = lax.fori_loop(0, _BSC, bbody, (zero,) * (2 * _NS),
                            unroll=4)
        for s in range(_NS):
            pstage[l, pl.ds(s * 16, 16)] = acc[s]
            tstage[l, pl.ds(s * 16, 16)] = acc[_NS + s]

    def pair(k, _):
        l = 2 * k
        pltpu.make_async_copy(x_hbm.at[pl.ds(_BSPLIT, _BSC), l0],
                              xbuf0, sem0).wait()
        compute(l, xbuf0)

        @pl.when(l + 2 < _LPW)
        def _pf0():
            pltpu.async_copy(x_hbm.at[pl.ds(_BSPLIT, _BSC), l0 + l + 2],
                             xbuf0, sem0)

        pltpu.make_async_copy(x_hbm.at[pl.ds(_BSPLIT, _BSC), l0],
                              xbuf1, sem1).wait()
        compute(l + 1, xbuf1)

        @pl.when(l + 3 < _LPW)
        def _pf1():
            pltpu.async_copy(x_hbm.at[pl.ds(_BSPLIT, _BSC), l0 + l + 3],
                             xbuf1, sem1)

        return _

    lax.fori_loop(0, _LPW // 2, pair, None)

    pltpu.sync_copy(pstage, pos_hbm.at[pl.ds(l0, _LPW)])
    pltpu.sync_copy(tstage, tot_hbm.at[pl.ds(l0, _LPW)])


_sc_reduce = functools.partial(
    pl.kernel,
    out_type=[
        jax.ShapeDtypeStruct((_L, _D), jnp.float32),
        jax.ShapeDtypeStruct((_L, _D), jnp.float32),
    ],
    mesh=plsc.VectorSubcoreMesh(core_axis_name="c", subcore_axis_name="s"),
    scratch_types=[
        pltpu.VMEM((_BSC, _D), jnp.float32),
        pltpu.VMEM((_BSC, _D), jnp.float32),
        pltpu.VMEM((_LPW, _BSC), jnp.float32),
        pltpu.VMEM((_LPW, _D), jnp.float32),
        pltpu.VMEM((_LPW, _D), jnp.float32),
        pltpu.SemaphoreType.DMA,
        pltpu.SemaphoreType.DMA,
    ],
)(_sc_reduce_body)


def _tc_reduce_body(x_ref, mask_ref, pos_ref, tot_ref, pos_acc, tot_acc):
    i = pl.program_id(0)

    @pl.when(i == 0)
    def _init():
        pos_acc[...] = jnp.zeros_like(pos_acc)
        tot_acc[...] = jnp.zeros_like(tot_acc)

    xb = x_ref[...]
    mb = mask_ref[...].astype(jnp.float32)
    pos_acc[...] += jnp.sum(mb[:, :, None] * xb, axis=0)
    tot_acc[...] += jnp.sum(xb, axis=0)

    @pl.when(i == _TC_STEPS - 1)
    def _finish():
        pos_ref[...] = pos_acc[...]
        tot_ref[...] = tot_acc[...]


def _tc_reduce(x, label_mask):
    return pl.pallas_call(
        _tc_reduce_body,
        grid=(_TC_STEPS,),
        in_specs=[
            pl.BlockSpec((_NB, _L, _D), lambda i: (i, 0, 0)),
            pl.BlockSpec((_NB, _L), lambda i: (i, 0)),
        ],
        out_specs=[
            pl.BlockSpec((_L, _D), lambda i: (0, 0)),
            pl.BlockSpec((_L, _D), lambda i: (0, 0)),
        ],
        out_shape=[
            jax.ShapeDtypeStruct((_L, _D), jnp.float32),
            jax.ShapeDtypeStruct((_L, _D), jnp.float32),
        ],
        scratch_shapes=[
            pltpu.VMEM((_L, _D), jnp.float32),
            pltpu.VMEM((_L, _D), jnp.float32),
        ],
        compiler_params=pltpu.CompilerParams(
            dimension_semantics=("arbitrary",),
        ),
    )(x, label_mask)


def _mlp(v, wh, bh, wp, bp):
    h = jnp.maximum(
        lax.dot_general(v, wh, (((1,), (1,)), ((), ())),
                        preferred_element_type=jnp.float32) + bh,
        0.0)
    return lax.dot_general(h, wp, (((1,), (1,)), ((), ())),
                           preferred_element_type=jnp.float32) + bp


def _tc_finish_body(pos1_ref, tot1_ref, pos2_ref, tot2_ref, maskT_ref,
                    gpt_ref, gpt_last_ref, wh_ref, bh_ref, wp_ref, bp_ref,
                    out_ref):
    cnt = jnp.sum(maskT_ref[...], axis=1, keepdims=True)
    pos = pos1_ref[...] + pos2_ref[...]
    tot = tot1_ref[...] + tot2_ref[...]
    neg_cnt = _B - cnt
    pos_mean = pos / jnp.maximum(cnt, 1.0)
    neg_mean = (tot - pos) / jnp.maximum(neg_cnt, 1.0)
    avg = 0.5 * pos_mean + 0.5 * gpt_ref[...]
    avg_anti = 0.5 * neg_mean + 0.5 * gpt_last_ref[...]
    wh = wh_ref[...]
    bh = bh_ref[...]
    wp = wp_ref[...]
    bp = bp_ref[...]
    proto = _mlp(avg, wh, bh, wp, bp)
    proto = jnp.where(cnt > 0.0, proto, 0.0)
    anti = _mlp(avg_anti, wh, bh, wp, bp)
    valid = (neg_cnt > 0.0).astype(jnp.float32)
    anti_sum = jnp.sum(anti * valid, axis=0, keepdims=True)
    anti_row = anti_sum / jnp.maximum(jnp.sum(valid), 1.0)
    out_ref[0:_L, :] = proto
    out_ref[_L:_L + 1, :] = anti_row


def _tc_finish(pos1, tot1, pos2, tot2, maskT, gpt_main, gpt_last,
               wh, bh, wp, bp):
    return pl.pallas_call(
        _tc_finish_body,
        out_shape=jax.ShapeDtypeStruct((_L + 1, _D), jnp.float32),
    )(pos1, tot1, pos2, tot2, maskT, gpt_main, gpt_last, wh, bh, wp, bp)


@jax.jit
def _run(x, label_mask, gpt_main, gpt_last, wh, bh, wp, bp):
    maskT = jnp.transpose(label_mask, (1, 0)).astype(jnp.float32)
    pos_sc, tot_sc = _sc_reduce(x, maskT[:, _BSPLIT:])
    pos_tc, tot_tc = _tc_reduce(x, label_mask)
    return _tc_finish(pos_tc, tot_tc, pos_sc, tot_sc, maskT,
                      gpt_main, gpt_last, wh, bh, wp, bp)


def kernel(x, label_mask, global_prototype_tensor, W_hidden, b_hidden,
           W_proto, b_proto):
    gpt_main = global_prototype_tensor[:_L]
    gpt_last = global_prototype_tensor[_L:]
    return _run(x, label_mask, gpt_main, gpt_last, W_hidden,
                b_hidden.reshape(1, _H), W_proto, b_proto.reshape(1, _D))
